# baseline (speedup 1.0000x reference)
import functools
import jax
import jax.numpy as jnp
from jax import lax
from jax.experimental import pallas as pl
from jax.experimental.pallas import tpu as pltpu
from jax.experimental.pallas import tpu_sc as plsc

N = 10000
H = 128
NPAD = 10240
TCR = 512
NBLK = NPAD // TCR
SENT = NPAD - 1
PBITS = 24
RW = 512
NTILES = 32
NZCAP = 8192
CAP_K = 384
NCH = 6
CAPE = NCH * CAP_K
CWMAX = 16384
HH = 64
HT = 80
NEG_SLOPE = 0.2

PACK_R = 256
PACK_C = 3072
PACK_O = PACK_C // PBITS


def _pack_body(adj_ref, actr_ref, actc_ref, p_ref, out_ref):
    a = adj_ref[...]
    bits = jnp.where(
        (a > 0.0) & (actr_ref[...] > 0.0) & (actc_ref[...] > 0.0), 1.0, 0.0)
    out_ref[...] = lax.dot_general(
        bits.astype(jnp.bfloat16), p_ref[...],
        (((1,), (0,)), ((), ())), preferred_element_type=jnp.float32)


ROWS0 = 5120
ROWS1 = N - ROWS0
RB0 = ROWS0 // PACK_R


def _pack_call(adj, actr, actc, pmat, half):
    rows = ROWS0 if half == 0 else ROWS1
    rb = 0 if half == 0 else RB0
    grid = (pl.cdiv(rows, PACK_R), RW * PBITS // PACK_C)
    return pl.pallas_call(
        _pack_body,
        grid=grid,
        in_specs=[
            pl.BlockSpec((PACK_R, PACK_C), lambda r, c: (r + rb, c)),
            pl.BlockSpec((PACK_R, 1), lambda r, c: (r + rb, 0)),
            pl.BlockSpec((1, PACK_C), lambda r, c: (0, c)),
            pl.BlockSpec((PACK_C, PACK_O), lambda r, c: (0, 0)),
        ],
        out_specs=pl.BlockSpec((PACK_R, PACK_O), lambda r, c: (r, c)),
        out_shape=jax.ShapeDtypeStruct((rows, RW), jnp.float32),
    )(adj, actr, actc, pmat)


def _mk_extract_body(wpt, cw, nchunks, roff):
  def _extract_body(bytes_hbm, src_hbm, dst_hbm, cnt_hbm,
                    chunk_v, nzval_v, nzidx_v, srcbuf_v, dstbuf_v, out16_v,
                    sem):
    cid = lax.axis_index("c")
    sid = lax.axis_index("s")
    wid = sid * 2 + cid
    base = wid * wpt

    def _z(i, c):
        nzval_v[pl.ds(i * 16, 16)] = jnp.zeros((16,), jnp.float32)
        return c
    lax.fori_loop(0, NZCAP // 16, _z, 0)

    def _f(i, c):
        sent = N + ((i * 16) % (NPAD - N)) + lax.iota(jnp.int32, 16)
        srcbuf_v[pl.ds(i * 16, 16)] = sent
        dstbuf_v[pl.ds(i * 16, 16)] = sent
        return c
    lax.fori_loop(0, CAPE // 16, _f, 0)

    def _chunk(ch, ofs):
        pltpu.sync_copy(bytes_hbm.at[pl.ds(base + ch * cw, cw)],
                        chunk_v.at[pl.ds(0, cw)])

        def _grp(j, o):
            vs = [chunk_v[pl.ds((j * 4 + t) * 16, 16)] for t in range(4)]
            mx = jnp.maximum(jnp.maximum(vs[0], vs[1]),
                             jnp.maximum(vs[2], vs[3]))
            anynz = jnp.max(mx) > 0.0

            def _do(oo):
                for t in range(4):
                    v = vs[t]
                    m = v != 0.0
                    oo = jnp.minimum(oo, NZCAP - 16)
                    plsc.store_compressed(nzval_v.at[pl.ds(oo, 16)], v, mask=m)
                    idxv = (base + ch * cw + (j * 4 + t) * 16
                            + lax.iota(jnp.int32, 16)).astype(jnp.int32)
                    plsc.store_compressed(nzidx_v.at[pl.ds(oo, 16)], idxv,
                                          mask=m)
                    oo = oo + jnp.sum(m.astype(jnp.int32))
                return oo
            return lax.cond(anynz, _do, lambda oo: oo, o)
        return lax.fori_loop(0, cw // 64, _grp, ofs)
    nzcnt = lax.fori_loop(0, nchunks, _chunk, jnp.int32(0))

    def _dec(q, eo):
        wv = nzval_v[pl.ds(q * 16, 16)]
        wi = nzidx_v[pl.ds(q * 16, 16)]
        w = wv.astype(jnp.int32)
        srcv = wi // RW + roff
        dstb = (wi % RW) * PBITS
        for b in range(PBITS):
            mb = ((w >> b) & 1) != 0
            eo = jnp.minimum(eo, CAPE - 16)
            plsc.store_compressed(srcbuf_v.at[pl.ds(eo, 16)], srcv, mask=mb)
            plsc.store_compressed(dstbuf_v.at[pl.ds(eo, 16)], dstb + b, mask=mb)
            eo = eo + jnp.sum(mb.astype(jnp.int32))
        return eo
    ecnt = lax.fori_loop(0, pl.cdiv(nzcnt, 16), _dec, jnp.int32(0))

    def _w(ch, c):
        pltpu.sync_copy(srcbuf_v.at[pl.ds(ch * CAP_K, CAP_K)],
                        src_hbm.at[wid, ch])
        pltpu.sync_copy(dstbuf_v.at[pl.ds(ch * CAP_K, CAP_K)],
                        dst_hbm.at[wid, ch])
        return c
    lax.fori_loop(0, NCH, _w, 0)
    out16_v[...] = jnp.broadcast_to(ecnt, (16,)).astype(jnp.int32)
    pltpu.sync_copy(out16_v, cnt_hbm.at[wid])
  return _extract_body


def _extract_call(bytes_flat, half):
    nw = bytes_flat.shape[0]
    wpt = nw // NTILES
    cw = 16384 if half == 0 else 15616
    roff = 0 if half == 0 else ROWS0
    mesh = plsc.VectorSubcoreMesh(core_axis_name="c", subcore_axis_name="s")
    f = pl.kernel(
        _mk_extract_body(wpt, cw, wpt // cw, roff),
        mesh=mesh,
        out_type=[
            jax.ShapeDtypeStruct((NTILES, NCH, CAP_K), jnp.int32),
            jax.ShapeDtypeStruct((NTILES, NCH, CAP_K), jnp.int32),
            jax.ShapeDtypeStruct((NTILES, 16), jnp.int32),
        ],
        scratch_types=[
            pltpu.VMEM((CWMAX,), jnp.float32),
            pltpu.VMEM((NZCAP,), jnp.float32),
            pltpu.VMEM((NZCAP,), jnp.int32),
            pltpu.VMEM((CAPE,), jnp.int32),
            pltpu.VMEM((CAPE,), jnp.int32),
            pltpu.VMEM((16,), jnp.int32),
            pltpu.SemaphoreType.DMA,
        ],
        compiler_params=pltpu.CompilerParams(
            needs_layout_passes=False, use_tc_tiling_on_sc=False),
    )
    return f(bytes_flat)


def _msg_body(htab_hbm, as_hbm, ad_hbm, gmax_hbm, src0_hbm, dst0_hbm,
              cnt0_hbm, src1_hbm, dst1_hbm, cnt1_hbm,
              acc_out,
              vm_as, vm_ad, vm_g, src2d, dst2d, cnt16, rows_v, pbuf,
              zerob, acc_sh, sem):
    cid = lax.axis_index("c")
    sid = lax.axis_index("s")

    pltpu.sync_copy(as_hbm, vm_as)
    pltpu.sync_copy(ad_hbm, vm_ad)
    pltpu.sync_copy(gmax_hbm.at[pl.ds(0, 16)], vm_g)
    g = vm_g[...][0]

    def _zb(i, c):
        for k in range(HT // 16):
            zerob[i, pl.ds(k * 16, 16)] = jnp.zeros((16,), jnp.float32)
        return c
    lax.fori_loop(0, 64, _zb, 0)
    rows_per = NPAD // 16

    def _zs(r, c):
        pltpu.sync_copy(zerob, acc_sh.at[pl.ds(sid * rows_per + r * 64, 64), :])
        return c
    lax.fori_loop(0, rows_per // 64, _zs, 0)
    plsc.subcore_barrier()

    for src_hbm, dst_hbm, cnt_hbm in ((src0_hbm, dst0_hbm, cnt0_hbm),
                                      (src1_hbm, dst1_hbm, cnt1_hbm)):
      for seg in range(2):
        wid = sid * 2 + seg
        pltpu.sync_copy(cnt_hbm.at[wid], cnt16)
        cnt = cnt16[...][0]
        pltpu.sync_copy(src_hbm.at[wid], src2d)
        pltpu.sync_copy(dst_hbm.at[wid], dst2d)

        def _chunk(ch, c):
            pltpu.async_copy(htab_hbm.at[cid].at[src2d.at[ch]], rows_v,
                             sem).wait()
            for i in range(CAP_K // 16):
                sv = src2d[ch, pl.ds(i * 16, 16)]
                dv = dst2d[ch, pl.ds(i * 16, 16)]
                asg = plsc.load_gather(vm_as, [sv])
                adg = plsc.load_gather(vm_ad, [dv])
                mg = g + adg
                mg = jnp.where(mg >= 0.0, mg, NEG_SLOPE * mg)
                z = asg + adg
                z = jnp.where(z >= 0.0, z, NEG_SLOPE * z)
                pbuf[pl.ds(i * 16, 16)] = jnp.exp(z - mg)

            def _scale(q, cc):
                p16 = pbuf[pl.ds(q * 16, 16)]
                for i in range(16):
                    r = q * 16 + i
                    pr = p16[i]
                    for k in range(HT // 16):
                        rows_v[r, pl.ds(k * 16, 16)] = (
                            rows_v[r, pl.ds(k * 16, 16)] * pr)
                return cc
            lax.fori_loop(0, CAP_K // 16, _scale, 0)
            pltpu.sync_copy(rows_v, acc_sh.at[dst2d.at[ch]], add=True)
            return c
        lax.fori_loop(0, pl.cdiv(cnt, CAP_K), _chunk, 0)

    plsc.subcore_barrier()
    pltpu.sync_copy(acc_sh.at[pl.ds(sid * rows_per, rows_per), :],
                    acc_out.at[cid, pl.ds(sid * rows_per, rows_per), :])


def _msg_call(htab, asf, adf, gmaxf, src0, dst0, cnt0, src1, dst1, cnt1):
    mesh = plsc.VectorSubcoreMesh(core_axis_name="c", subcore_axis_name="s")
    f = pl.kernel(
        _msg_body,
        mesh=mesh,
        out_type=[jax.ShapeDtypeStruct((2, NPAD, HT), jnp.float32)],
        scratch_types=[
            pltpu.VMEM((NPAD,), jnp.float32),
            pltpu.VMEM((NPAD,), jnp.float32),
            pltpu.VMEM((16,), jnp.float32),
            pltpu.VMEM((NCH, CAP_K), jnp.int32),
            pltpu.VMEM((NCH, CAP_K), jnp.int32),
            pltpu.VMEM((16,), jnp.int32),
            pltpu.VMEM((CAP_K, HT), jnp.float32),
            pltpu.VMEM((CAP_K,), jnp.float32),
            pltpu.VMEM((64, HT), jnp.float32),
            pltpu.VMEM_SHARED((NPAD, HT), jnp.float32),
            pltpu.SemaphoreType.DMA,
        ],
        compiler_params=pltpu.CompilerParams(
            needs_layout_passes=False, use_tc_tiling_on_sc=False),
    )
    (acc,) = f(htab, asf, adf, gmaxf, src0, dst0, cnt0, src1, dst1, cnt1)
    return acc


def _head(x, gw_ref, atts_ref, attd_ref, b, htab_ref, as_ref, ad_ref,
          gmax_ref, rowmask):
    h = lax.dot_general(x, gw_ref[...], (((1,), (1,)), ((), ())),
                        preferred_element_type=jnp.float32)
    a_s = jnp.sum(h * atts_ref[...], axis=1, keepdims=True)
    a_d = jnp.sum(h * attd_ref[...], axis=1, keepdims=True)
    onescol = rowmask.astype(jnp.float32)
    zpad = jnp.zeros((TCR, HT - HH - 1), jnp.float32)
    htab_ref[0] = jnp.concatenate([h[:, 0:HH], onescol, zpad], axis=1)
    htab_ref[1] = jnp.concatenate([h[:, HH:H], onescol, zpad], axis=1)
    as_ref[...] = a_s
    ad_ref[...] = a_d

    @pl.when(b == 0)
    def _():
        gmax_ref[...] = jnp.full((1, H), -jnp.inf, jnp.float32)
    gmax_ref[...] = jnp.maximum(gmax_ref[...], jnp.max(a_s))


def _a0_body(prog_ref, hard_ref, w1_ref, b1_ref, w2_ref, b2_ref,
             gw_ref, atts_ref, attd_ref,
             x0_ref, htab_ref, as_ref, ad_ref, gmax_ref):
    b = pl.program_id(0)
    in2 = jnp.concatenate([prog_ref[...], hard_ref[...]], axis=1)
    t1 = lax.dot_general(in2, w1_ref[...], (((1,), (1,)), ((), ())),
                         preferred_element_type=jnp.float32) + b1_ref[...]
    x0 = lax.dot_general(t1, w2_ref[...], (((1,), (1,)), ((), ())),
                         preferred_element_type=jnp.float32) + b2_ref[...]
    rowmask = (lax.broadcasted_iota(jnp.int32, (TCR, 1), 0) + b * TCR) < N
    x0 = jnp.where(rowmask, x0, 0.0)
    x0_ref[...] = x0
    _head(x0, gw_ref, atts_ref, attd_ref, b, htab_ref, as_ref, ad_ref,
          gmax_ref, rowmask)


def _combine(acc_ref, htabp_ref, asp_ref, adp_ref, gmaxp_ref, bias_ref,
             apply_relu, rowmask):
    num = jnp.concatenate([acc_ref[0][:, 0:HH], acc_ref[1][:, 0:HH]], axis=1)
    den = acc_ref[0][:, HH:HH + 1]
    hprev = jnp.concatenate([htabp_ref[0][:, 0:HH], htabp_ref[1][:, 0:HH]],
                            axis=1)
    asb = asp_ref[...]
    adb = adp_ref[...]
    g = gmaxp_ref[0, 0]
    m = g + adb
    mm = jnp.where(m >= 0.0, m, NEG_SLOPE * m)
    zs = asb + adb
    zs = jnp.where(zs >= 0.0, zs, NEG_SLOPE * zs)
    p_self = jnp.exp(zs - mm)
    out = (num + p_self * hprev) / (den + p_self + 1e-16) + bias_ref[...]
    if apply_relu:
        out = jnp.maximum(out, 0.0)
    return jnp.where(rowmask, out, 0.0)


def _mk_anext_body(apply_relu):
    def _body(acc_ref, htabp_ref, asp_ref, adp_ref, gmaxp_ref, bias_ref,
              gw_ref, atts_ref, attd_ref,
              htab_ref, as_ref, ad_ref, gmax_ref):
        b = pl.program_id(0)
        rowmask = (lax.broadcasted_iota(jnp.int32, (TCR, 1), 0) + b * TCR) < N
        x = _combine(acc_ref, htabp_ref, asp_ref, adp_ref, gmaxp_ref,
                     bias_ref, apply_relu, rowmask)
        _head(x, gw_ref, atts_ref, attd_ref, b, htab_ref, as_ref, ad_ref,
              gmax_ref, rowmask)
    return _body


def _fin_body(acc_ref, htabp_ref, asp_ref, adp_ref, gmaxp_ref, bias_ref,
              x0_ref, act_ref, vw_ref, vb_ref,
              pooled_ref, na_ref, v_ref):
    b = pl.program_id(0)
    rowmask = (lax.broadcasted_iota(jnp.int32, (TCR, 1), 0) + b * TCR) < N
    gat = _combine(acc_ref, htabp_ref, asp_ref, adp_ref, gmaxp_ref,
                   bias_ref, False, rowmask)
    xf = x0_ref[...] + gat
    mu = jnp.mean(xf, axis=1, keepdims=True)
    var = jnp.mean((xf - mu) * (xf - mu), axis=1, keepdims=True)
    xn = (xf - mu) / jnp.sqrt(var + 1e-5)
    actb = act_ref[...]

    @pl.when(b == 0)
    def _():
        pooled_ref[...] = jnp.zeros((1, H), jnp.float32)
        na_ref[...] = jnp.zeros((1, H), jnp.float32)
    pooled_ref[...] = pooled_ref[...] + jnp.sum(xn * actb, axis=0,
                                                keepdims=True)
    na_ref[...] = na_ref[...] + jnp.sum(actb)

    @pl.when(b == NBLK - 1)
    def _():
        pooled = pooled_ref[...] / na_ref[0, 0]
        v = jnp.sum(pooled * vw_ref[...]) + vb_ref[0, 0]
        v_ref[...] = jnp.maximum(v, 0.0).reshape(1, 1)


_SPEC_COL = pl.BlockSpec((TCR, 1), lambda b: (b, 0))
_SPEC_ROW = pl.BlockSpec((TCR, H), lambda b: (b, 0))
_SPEC_HTAB = pl.BlockSpec((2, TCR, HT), lambda b: (0, b, 0))
_SPEC_FULL = pl.BlockSpec((H, H), lambda b: (0, 0))
_SPEC_VEC = pl.BlockSpec((1, H), lambda b: (0, 0))
_SPEC_ACC = pl.BlockSpec((2, TCR, HT), lambda b: (0, b, 0))

_HEAD_OUT_SPECS = [_SPEC_HTAB, _SPEC_COL, _SPEC_COL, _SPEC_VEC]


def _head_out_shapes():
    return [
        jax.ShapeDtypeStruct((2, NPAD, HT), jnp.float32),
        jax.ShapeDtypeStruct((NPAD, 1), jnp.float32),
        jax.ShapeDtypeStruct((NPAD, 1), jnp.float32),
        jax.ShapeDtypeStruct((1, H), jnp.float32),
    ]


def _a0_call(prog, hard, w1, b1, w2, b2, gw, atts, attd):
    return pl.pallas_call(
        _a0_body,
        grid=(NBLK,),
        in_specs=[_SPEC_COL, _SPEC_COL,
                  pl.BlockSpec((H, 2), lambda b: (0, 0)), _SPEC_VEC,
                  _SPEC_FULL, _SPEC_VEC, _SPEC_FULL, _SPEC_VEC, _SPEC_VEC],
        out_specs=[_SPEC_ROW] + _HEAD_OUT_SPECS,
        out_shape=[jax.ShapeDtypeStruct((NPAD, H), jnp.float32)]
        + _head_out_shapes(),
    )(prog, hard, w1, b1, w2, b2, gw, atts, attd)


def _anext_call(apply_relu, acc, htabp, asp, adp, gmaxp, bias, gw, atts, attd):
    return pl.pallas_call(
        _mk_anext_body(apply_relu),
        grid=(NBLK,),
        in_specs=[_SPEC_ACC, _SPEC_HTAB, _SPEC_COL, _SPEC_COL, _SPEC_VEC,
                  _SPEC_VEC, _SPEC_FULL, _SPEC_VEC, _SPEC_VEC],
        out_specs=_HEAD_OUT_SPECS,
        out_shape=_head_out_shapes(),
    )(acc, htabp, asp, adp, gmaxp, bias, gw, atts, attd)


def _fin_call(acc, htabp, asp, adp, gmaxp, bias, x0, act, vw, vb):
    return pl.pallas_call(
        _fin_body,
        grid=(NBLK,),
        in_specs=[_SPEC_ACC, _SPEC_HTAB, _SPEC_COL, _SPEC_COL, _SPEC_VEC,
                  _SPEC_VEC, _SPEC_ROW, _SPEC_COL, _SPEC_VEC,
                  pl.BlockSpec((1, 1), lambda b: (0, 0))],
        out_specs=[_SPEC_VEC, _SPEC_VEC, pl.BlockSpec((1, 1), lambda b: (0, 0))],
        out_shape=[
            jax.ShapeDtypeStruct((1, H), jnp.float32),
            jax.ShapeDtypeStruct((1, H), jnp.float32),
            jax.ShapeDtypeStruct((1, 1), jnp.float32),
        ],
    )(acc, htabp, asp, adp, gmaxp, bias, x0, act, vw, vb)


def _padcol(a):
    return jnp.pad(a.astype(jnp.float32), (0, NPAD - N)).reshape(NPAD, 1)


def kernel(adjacency_matrix, timestep, arrivals, departures, is_hard_to_match,
           active_agents, emb_W1, emb_b1, emb_W2, emb_b2, gat_W, gat_att_src,
           gat_att_dst, gat_bias, value_W, value_b):
    t = jnp.asarray(timestep, jnp.float32)
    prog = (t - arrivals) / (departures - arrivals)
    progc = _padcol(prog)
    hardc = _padcol(is_hard_to_match)
    actc = _padcol(active_agents)
    actr = active_agents.reshape(N, 1)
    actcol = jnp.pad(active_agents, (0, RW * PBITS - N)).reshape(1, RW * PBITS)

    ki = lax.broadcasted_iota(jnp.int32, (PACK_C, PACK_O), 0)
    gi = lax.broadcasted_iota(jnp.int32, (PACK_C, PACK_O), 1)
    pmat = jnp.where(ki // PBITS == gi,
                     (1 << (ki % PBITS)).astype(jnp.float32),
                     0.0).astype(jnp.bfloat16)

    bytes0 = _pack_call(adjacency_matrix, actr, actcol, pmat, 0)
    src0, dst0, cnt0 = _extract_call(bytes0.reshape(-1), 0)
    bytes1 = _pack_call(adjacency_matrix, actr, actcol, pmat, 1)
    src1, dst1, cnt1 = _extract_call(bytes1.reshape(-1), 1)

    x0, htab0, as0, ad0, g0 = _a0_call(
        progc, hardc, emb_W1, emb_b1.reshape(1, H), emb_W2,
        emb_b2.reshape(1, H), gat_W[0], gat_att_src[0].reshape(1, H),
        gat_att_dst[0].reshape(1, H))

    htab, asp, adp, gp = htab0, as0, ad0, g0
    for l in range(3):
        acc = _msg_call(htab, asp.reshape(NPAD), adp.reshape(NPAD),
                        gp.reshape(H), src0, dst0, cnt0, src1, dst1, cnt1)
        bias = gat_bias[l].reshape(1, H)
        if l < 2:
            htab, asp, adp, gp = _anext_call(
                True, acc, htab, asp, adp, gp, bias, gat_W[l + 1],
                gat_att_src[l + 1].reshape(1, H),
                gat_att_dst[l + 1].reshape(1, H))
        else:
            _, _, v = _fin_call(acc, htab, asp, adp, gp, bias, x0, actc,
                                value_W.reshape(1, H), value_b.reshape(1, 1))
    return v[0, 0]


# speedup vs baseline: 1.0292x; 1.0292x over previous
import functools
import jax
import jax.numpy as jnp
from jax import lax
from jax.experimental import pallas as pl
from jax.experimental.pallas import tpu as pltpu
from jax.experimental.pallas import tpu_sc as plsc

N = 10000
H = 128
NPAD = 10240
TCR = 512
NBLK = NPAD // TCR
SENT = NPAD - 1
PBITS = 24
RW = 512
NTILES = 32
NZCAP = 8192
CAP_K = 384
NCH = 11
CAPE = NCH * CAP_K
CWMAX = 16384
HH = 64
HT = 80
NEG_SLOPE = 0.2

PACK_R = 256
PACK_C = 3072
PACK_O = PACK_C // PBITS


def _pack_body(adj_ref, actr_ref, actc_ref, p_ref, out_ref):
    a = adj_ref[...]
    bits = jnp.where(
        (a > 0.0) & (actr_ref[...] > 0.0) & (actc_ref[...] > 0.0), 1.0, 0.0)
    out_ref[...] = lax.dot_general(
        bits.astype(jnp.bfloat16), p_ref[...],
        (((1,), (0,)), ((), ())), preferred_element_type=jnp.float32)


def _pack_call(adj, actr, actc, pmat):
    grid = (pl.cdiv(N, PACK_R), RW * PBITS // PACK_C)
    return pl.pallas_call(
        _pack_body,
        grid=grid,
        in_specs=[
            pl.BlockSpec((PACK_R, PACK_C), lambda r, c: (r, c)),
            pl.BlockSpec((PACK_R, 1), lambda r, c: (r, 0)),
            pl.BlockSpec((1, PACK_C), lambda r, c: (0, c)),
            pl.BlockSpec((PACK_C, PACK_O), lambda r, c: (0, 0)),
        ],
        out_specs=pl.BlockSpec((PACK_R, PACK_O), lambda r, c: (r, c)),
        out_shape=jax.ShapeDtypeStruct((N, RW), jnp.float32),
    )(adj, actr, actc, pmat)


def _mk_extract_body(wpt, cw, nchunks, roff):
  def _extract_body(bytes_hbm, src_hbm, dst_hbm, cnt_hbm,
                    chunk_v, nzval_v, nzidx_v, srcbuf_v, dstbuf_v, out16_v,
                    sem):
    cid = lax.axis_index("c")
    sid = lax.axis_index("s")
    wid = sid * 2 + cid
    base = wid * wpt

    def _z(i, c):
        nzval_v[pl.ds(i * 16, 16)] = jnp.zeros((16,), jnp.float32)
        return c
    lax.fori_loop(0, NZCAP // 16, _z, 0)

    def _f(i, c):
        sent = N + ((i * 16) % (NPAD - N)) + lax.iota(jnp.int32, 16)
        srcbuf_v[pl.ds(i * 16, 16)] = sent
        dstbuf_v[pl.ds(i * 16, 16)] = sent
        return c
    lax.fori_loop(0, CAPE // 16, _f, 0)

    def _chunk(ch, ofs):
        pltpu.sync_copy(bytes_hbm.at[pl.ds(base + ch * cw, cw)],
                        chunk_v.at[pl.ds(0, cw)])

        def _grp(j, o):
            vs = [chunk_v[pl.ds((j * 4 + t) * 16, 16)] for t in range(4)]
            mx = jnp.maximum(jnp.maximum(vs[0], vs[1]),
                             jnp.maximum(vs[2], vs[3]))
            anynz = jnp.max(mx) > 0.0

            def _do(oo):
                for t in range(4):
                    v = vs[t]
                    m = v != 0.0
                    oo = jnp.minimum(oo, NZCAP - 16)
                    plsc.store_compressed(nzval_v.at[pl.ds(oo, 16)], v, mask=m)
                    idxv = (base + ch * cw + (j * 4 + t) * 16
                            + lax.iota(jnp.int32, 16)).astype(jnp.int32)
                    plsc.store_compressed(nzidx_v.at[pl.ds(oo, 16)], idxv,
                                          mask=m)
                    oo = oo + jnp.sum(m.astype(jnp.int32))
                return oo
            return lax.cond(anynz, _do, lambda oo: oo, o)
        return lax.fori_loop(0, cw // 64, _grp, ofs)
    nzcnt = lax.fori_loop(0, nchunks, _chunk, jnp.int32(0))

    def _dec(q, eo):
        wv = nzval_v[pl.ds(q * 16, 16)]
        wi = nzidx_v[pl.ds(q * 16, 16)]
        w = wv.astype(jnp.int32)
        srcv = wi // RW + roff
        dstb = (wi % RW) * PBITS
        for b in range(PBITS):
            mb = ((w >> b) & 1) != 0
            eo = jnp.minimum(eo, CAPE - 16)
            plsc.store_compressed(srcbuf_v.at[pl.ds(eo, 16)], srcv, mask=mb)
            plsc.store_compressed(dstbuf_v.at[pl.ds(eo, 16)], dstb + b, mask=mb)
            eo = eo + jnp.sum(mb.astype(jnp.int32))
        return eo
    ecnt = lax.fori_loop(0, pl.cdiv(nzcnt, 16), _dec, jnp.int32(0))

    def _w(ch, c):
        pltpu.sync_copy(srcbuf_v.at[pl.ds(ch * CAP_K, CAP_K)],
                        src_hbm.at[wid, ch])
        pltpu.sync_copy(dstbuf_v.at[pl.ds(ch * CAP_K, CAP_K)],
                        dst_hbm.at[wid, ch])
        return c
    lax.fori_loop(0, NCH, _w, 0)
    out16_v[...] = jnp.broadcast_to(ecnt, (16,)).astype(jnp.int32)
    pltpu.sync_copy(out16_v, cnt_hbm.at[wid])
  return _extract_body


def _extract_call(bytes_flat):
    wpt = bytes_flat.shape[0] // NTILES
    cw = 16000
    mesh = plsc.VectorSubcoreMesh(core_axis_name="c", subcore_axis_name="s")
    f = pl.kernel(
        _mk_extract_body(wpt, cw, wpt // cw, 0),
        mesh=mesh,
        out_type=[
            jax.ShapeDtypeStruct((NTILES, NCH, CAP_K), jnp.int32),
            jax.ShapeDtypeStruct((NTILES, NCH, CAP_K), jnp.int32),
            jax.ShapeDtypeStruct((NTILES, 16), jnp.int32),
        ],
        scratch_types=[
            pltpu.VMEM((CWMAX,), jnp.float32),
            pltpu.VMEM((NZCAP,), jnp.float32),
            pltpu.VMEM((NZCAP,), jnp.int32),
            pltpu.VMEM((CAPE,), jnp.int32),
            pltpu.VMEM((CAPE,), jnp.int32),
            pltpu.VMEM((16,), jnp.int32),
            pltpu.SemaphoreType.DMA,
        ],
        compiler_params=pltpu.CompilerParams(
            needs_layout_passes=False, use_tc_tiling_on_sc=False),
    )
    return f(bytes_flat)


def _msg_body(htab_hbm, as_hbm, ad_hbm, gmax_hbm, src0_hbm, dst0_hbm,
              cnt0_hbm,
              acc_out,
              vm_as, vm_ad, vm_g, src2d, dst2d, cnt16, rows_v, pbuf,
              zerob, acc_sh, sem):
    cid = lax.axis_index("c")
    sid = lax.axis_index("s")

    pltpu.sync_copy(as_hbm, vm_as)
    pltpu.sync_copy(ad_hbm, vm_ad)
    pltpu.sync_copy(gmax_hbm.at[pl.ds(0, 16)], vm_g)
    g = vm_g[...][0]

    def _zb(i, c):
        for k in range(HT // 16):
            zerob[i, pl.ds(k * 16, 16)] = jnp.zeros((16,), jnp.float32)
        return c
    lax.fori_loop(0, 64, _zb, 0)
    rows_per = NPAD // 16

    def _zs(r, c):
        pltpu.sync_copy(zerob, acc_sh.at[pl.ds(sid * rows_per + r * 64, 64), :])
        return c
    lax.fori_loop(0, rows_per // 64, _zs, 0)
    plsc.subcore_barrier()

    for src_hbm, dst_hbm, cnt_hbm in ((src0_hbm, dst0_hbm, cnt0_hbm),):
      for seg in range(2):
        wid = sid * 2 + seg
        pltpu.sync_copy(cnt_hbm.at[wid], cnt16)
        cnt = cnt16[...][0]
        pltpu.sync_copy(src_hbm.at[wid], src2d)
        pltpu.sync_copy(dst_hbm.at[wid], dst2d)

        def _chunk(ch, c):
            pltpu.async_copy(htab_hbm.at[cid].at[src2d.at[ch]], rows_v,
                             sem).wait()
            for i in range(CAP_K // 16):
                sv = src2d[ch, pl.ds(i * 16, 16)]
                dv = dst2d[ch, pl.ds(i * 16, 16)]
                asg = plsc.load_gather(vm_as, [sv])
                adg = plsc.load_gather(vm_ad, [dv])
                mg = g + adg
                mg = jnp.where(mg >= 0.0, mg, NEG_SLOPE * mg)
                z = asg + adg
                z = jnp.where(z >= 0.0, z, NEG_SLOPE * z)
                pbuf[pl.ds(i * 16, 16)] = jnp.exp(z - mg)

            def _scale(q, cc):
                p16 = pbuf[pl.ds(q * 16, 16)]
                for i in range(16):
                    r = q * 16 + i
                    pr = p16[i]
                    for k in range(HT // 16):
                        rows_v[r, pl.ds(k * 16, 16)] = (
                            rows_v[r, pl.ds(k * 16, 16)] * pr)
                return cc
            lax.fori_loop(0, CAP_K // 16, _scale, 0)
            pltpu.sync_copy(rows_v, acc_sh.at[dst2d.at[ch]], add=True)
            return c
        lax.fori_loop(0, pl.cdiv(cnt, CAP_K), _chunk, 0)

    plsc.subcore_barrier()
    pltpu.sync_copy(acc_sh.at[pl.ds(sid * rows_per, rows_per), :],
                    acc_out.at[cid, pl.ds(sid * rows_per, rows_per), :])


def _msg_call(htab, asf, adf, gmaxf, src0, dst0, cnt0):
    mesh = plsc.VectorSubcoreMesh(core_axis_name="c", subcore_axis_name="s")
    f = pl.kernel(
        _msg_body,
        mesh=mesh,
        out_type=[jax.ShapeDtypeStruct((2, NPAD, HT), jnp.float32)],
        scratch_types=[
            pltpu.VMEM((NPAD,), jnp.float32),
            pltpu.VMEM((NPAD,), jnp.float32),
            pltpu.VMEM((16,), jnp.float32),
            pltpu.VMEM((NCH, CAP_K), jnp.int32),
            pltpu.VMEM((NCH, CAP_K), jnp.int32),
            pltpu.VMEM((16,), jnp.int32),
            pltpu.VMEM((CAP_K, HT), jnp.float32),
            pltpu.VMEM((CAP_K,), jnp.float32),
            pltpu.VMEM((64, HT), jnp.float32),
            pltpu.VMEM_SHARED((NPAD, HT), jnp.float32),
            pltpu.SemaphoreType.DMA,
        ],
        compiler_params=pltpu.CompilerParams(
            needs_layout_passes=False, use_tc_tiling_on_sc=False),
    )
    (acc,) = f(htab, asf, adf, gmaxf, src0, dst0, cnt0)
    return acc


def _head(x, gw_ref, atts_ref, attd_ref, b, htab_ref, as_ref, ad_ref,
          gmax_ref, rowmask):
    h = lax.dot_general(x, gw_ref[...], (((1,), (1,)), ((), ())),
                        preferred_element_type=jnp.float32)
    a_s = jnp.sum(h * atts_ref[...], axis=1, keepdims=True)
    a_d = jnp.sum(h * attd_ref[...], axis=1, keepdims=True)
    onescol = rowmask.astype(jnp.float32)
    zpad = jnp.zeros((TCR, HT - HH - 1), jnp.float32)
    htab_ref[0] = jnp.concatenate([h[:, 0:HH], onescol, zpad], axis=1)
    htab_ref[1] = jnp.concatenate([h[:, HH:H], onescol, zpad], axis=1)
    as_ref[...] = a_s
    ad_ref[...] = a_d

    @pl.when(b == 0)
    def _():
        gmax_ref[...] = jnp.full((1, H), -jnp.inf, jnp.float32)
    gmax_ref[...] = jnp.maximum(gmax_ref[...], jnp.max(a_s))


def _a0_body(prog_ref, hard_ref, w1_ref, b1_ref, w2_ref, b2_ref,
             gw_ref, atts_ref, attd_ref,
             x0_ref, htab_ref, as_ref, ad_ref, gmax_ref):
    b = pl.program_id(0)
    in2 = jnp.concatenate([prog_ref[...], hard_ref[...]], axis=1)
    t1 = lax.dot_general(in2, w1_ref[...], (((1,), (1,)), ((), ())),
                         preferred_element_type=jnp.float32) + b1_ref[...]
    x0 = lax.dot_general(t1, w2_ref[...], (((1,), (1,)), ((), ())),
                         preferred_element_type=jnp.float32) + b2_ref[...]
    rowmask = (lax.broadcasted_iota(jnp.int32, (TCR, 1), 0) + b * TCR) < N
    x0 = jnp.where(rowmask, x0, 0.0)
    x0_ref[...] = x0
    _head(x0, gw_ref, atts_ref, attd_ref, b, htab_ref, as_ref, ad_ref,
          gmax_ref, rowmask)


def _combine(acc_ref, htabp_ref, asp_ref, adp_ref, gmaxp_ref, bias_ref,
             apply_relu, rowmask):
    num = jnp.concatenate([acc_ref[0][:, 0:HH], acc_ref[1][:, 0:HH]], axis=1)
    den = acc_ref[0][:, HH:HH + 1]
    hprev = jnp.concatenate([htabp_ref[0][:, 0:HH], htabp_ref[1][:, 0:HH]],
                            axis=1)
    asb = asp_ref[...]
    adb = adp_ref[...]
    g = gmaxp_ref[0, 0]
    m = g + adb
    mm = jnp.where(m >= 0.0, m, NEG_SLOPE * m)
    zs = asb + adb
    zs = jnp.where(zs >= 0.0, zs, NEG_SLOPE * zs)
    p_self = jnp.exp(zs - mm)
    out = (num + p_self * hprev) / (den + p_self + 1e-16) + bias_ref[...]
    if apply_relu:
        out = jnp.maximum(out, 0.0)
    return jnp.where(rowmask, out, 0.0)


def _mk_anext_body(apply_relu):
    def _body(acc_ref, htabp_ref, asp_ref, adp_ref, gmaxp_ref, bias_ref,
              gw_ref, atts_ref, attd_ref,
              htab_ref, as_ref, ad_ref, gmax_ref):
        b = pl.program_id(0)
        rowmask = (lax.broadcasted_iota(jnp.int32, (TCR, 1), 0) + b * TCR) < N
        x = _combine(acc_ref, htabp_ref, asp_ref, adp_ref, gmaxp_ref,
                     bias_ref, apply_relu, rowmask)
        _head(x, gw_ref, atts_ref, attd_ref, b, htab_ref, as_ref, ad_ref,
              gmax_ref, rowmask)
    return _body


def _fin_body(acc_ref, htabp_ref, asp_ref, adp_ref, gmaxp_ref, bias_ref,
              x0_ref, act_ref, vw_ref, vb_ref,
              pooled_ref, na_ref, v_ref):
    b = pl.program_id(0)
    rowmask = (lax.broadcasted_iota(jnp.int32, (TCR, 1), 0) + b * TCR) < N
    gat = _combine(acc_ref, htabp_ref, asp_ref, adp_ref, gmaxp_ref,
                   bias_ref, False, rowmask)
    xf = x0_ref[...] + gat
    mu = jnp.mean(xf, axis=1, keepdims=True)
    var = jnp.mean((xf - mu) * (xf - mu), axis=1, keepdims=True)
    xn = (xf - mu) / jnp.sqrt(var + 1e-5)
    actb = act_ref[...]

    @pl.when(b == 0)
    def _():
        pooled_ref[...] = jnp.zeros((1, H), jnp.float32)
        na_ref[...] = jnp.zeros((1, H), jnp.float32)
    pooled_ref[...] = pooled_ref[...] + jnp.sum(xn * actb, axis=0,
                                                keepdims=True)
    na_ref[...] = na_ref[...] + jnp.sum(actb)

    @pl.when(b == NBLK - 1)
    def _():
        pooled = pooled_ref[...] / na_ref[0, 0]
        v = jnp.sum(pooled * vw_ref[...]) + vb_ref[0, 0]
        v_ref[...] = jnp.maximum(v, 0.0).reshape(1, 1)


_SPEC_COL = pl.BlockSpec((TCR, 1), lambda b: (b, 0))
_SPEC_ROW = pl.BlockSpec((TCR, H), lambda b: (b, 0))
_SPEC_HTAB = pl.BlockSpec((2, TCR, HT), lambda b: (0, b, 0))
_SPEC_FULL = pl.BlockSpec((H, H), lambda b: (0, 0))
_SPEC_VEC = pl.BlockSpec((1, H), lambda b: (0, 0))
_SPEC_ACC = pl.BlockSpec((2, TCR, HT), lambda b: (0, b, 0))

_HEAD_OUT_SPECS = [_SPEC_HTAB, _SPEC_COL, _SPEC_COL, _SPEC_VEC]


def _head_out_shapes():
    return [
        jax.ShapeDtypeStruct((2, NPAD, HT), jnp.float32),
        jax.ShapeDtypeStruct((NPAD, 1), jnp.float32),
        jax.ShapeDtypeStruct((NPAD, 1), jnp.float32),
        jax.ShapeDtypeStruct((1, H), jnp.float32),
    ]


def _a0_call(prog, hard, w1, b1, w2, b2, gw, atts, attd):
    return pl.pallas_call(
        _a0_body,
        grid=(NBLK,),
        in_specs=[_SPEC_COL, _SPEC_COL,
                  pl.BlockSpec((H, 2), lambda b: (0, 0)), _SPEC_VEC,
                  _SPEC_FULL, _SPEC_VEC, _SPEC_FULL, _SPEC_VEC, _SPEC_VEC],
        out_specs=[_SPEC_ROW] + _HEAD_OUT_SPECS,
        out_shape=[jax.ShapeDtypeStruct((NPAD, H), jnp.float32)]
        + _head_out_shapes(),
    )(prog, hard, w1, b1, w2, b2, gw, atts, attd)


def _anext_call(apply_relu, acc, htabp, asp, adp, gmaxp, bias, gw, atts, attd):
    return pl.pallas_call(
        _mk_anext_body(apply_relu),
        grid=(NBLK,),
        in_specs=[_SPEC_ACC, _SPEC_HTAB, _SPEC_COL, _SPEC_COL, _SPEC_VEC,
                  _SPEC_VEC, _SPEC_FULL, _SPEC_VEC, _SPEC_VEC],
        out_specs=_HEAD_OUT_SPECS,
        out_shape=_head_out_shapes(),
    )(acc, htabp, asp, adp, gmaxp, bias, gw, atts, attd)


def _fin_call(acc, htabp, asp, adp, gmaxp, bias, x0, act, vw, vb):
    return pl.pallas_call(
        _fin_body,
        grid=(NBLK,),
        in_specs=[_SPEC_ACC, _SPEC_HTAB, _SPEC_COL, _SPEC_COL, _SPEC_VEC,
                  _SPEC_VEC, _SPEC_ROW, _SPEC_COL, _SPEC_VEC,
                  pl.BlockSpec((1, 1), lambda b: (0, 0))],
        out_specs=[_SPEC_VEC, _SPEC_VEC, pl.BlockSpec((1, 1), lambda b: (0, 0))],
        out_shape=[
            jax.ShapeDtypeStruct((1, H), jnp.float32),
            jax.ShapeDtypeStruct((1, H), jnp.float32),
            jax.ShapeDtypeStruct((1, 1), jnp.float32),
        ],
    )(acc, htabp, asp, adp, gmaxp, bias, x0, act, vw, vb)


def _padcol(a):
    return jnp.pad(a.astype(jnp.float32), (0, NPAD - N)).reshape(NPAD, 1)


def kernel(adjacency_matrix, timestep, arrivals, departures, is_hard_to_match,
           active_agents, emb_W1, emb_b1, emb_W2, emb_b2, gat_W, gat_att_src,
           gat_att_dst, gat_bias, value_W, value_b):
    t = jnp.asarray(timestep, jnp.float32)
    prog = (t - arrivals) / (departures - arrivals)
    progc = _padcol(prog)
    hardc = _padcol(is_hard_to_match)
    actc = _padcol(active_agents)
    actr = active_agents.reshape(N, 1)
    actcol = jnp.pad(active_agents, (0, RW * PBITS - N)).reshape(1, RW * PBITS)

    ki = lax.broadcasted_iota(jnp.int32, (PACK_C, PACK_O), 0)
    gi = lax.broadcasted_iota(jnp.int32, (PACK_C, PACK_O), 1)
    pmat = jnp.where(ki // PBITS == gi,
                     (1 << (ki % PBITS)).astype(jnp.float32),
                     0.0).astype(jnp.bfloat16)

    bytes0 = _pack_call(adjacency_matrix, actr, actcol, pmat)
    src0, dst0, cnt0 = _extract_call(bytes0.reshape(-1))

    x0, htab0, as0, ad0, g0 = _a0_call(
        progc, hardc, emb_W1, emb_b1.reshape(1, H), emb_W2,
        emb_b2.reshape(1, H), gat_W[0], gat_att_src[0].reshape(1, H),
        gat_att_dst[0].reshape(1, H))

    htab, asp, adp, gp = htab0, as0, ad0, g0
    for l in range(3):
        acc = _msg_call(htab, asp.reshape(NPAD), adp.reshape(NPAD),
                        gp.reshape(H), src0, dst0, cnt0)
        bias = gat_bias[l].reshape(1, H)
        if l < 2:
            htab, asp, adp, gp = _anext_call(
                True, acc, htab, asp, adp, gp, bias, gat_W[l + 1],
                gat_att_src[l + 1].reshape(1, H),
                gat_att_dst[l + 1].reshape(1, H))
        else:
            _, _, v = _fin_call(acc, htab, asp, adp, gp, bias, x0, actc,
                                value_W.reshape(1, H), value_b.reshape(1, 1))
    return v[0, 0]


# speedup vs baseline: 1.0428x; 1.0133x over previous
import functools
import jax
import jax.numpy as jnp
from jax import lax
from jax.experimental import pallas as pl
from jax.experimental.pallas import tpu as pltpu
from jax.experimental.pallas import tpu_sc as plsc

N = 10000
H = 128
NPAD = 10240
TCR = 512
NBLK = NPAD // TCR
SENT = NPAD - 1
PBITS = 24
RW = 512
NTILES = 32
NZCAP = 8192
CAP_K = 384
NCH = 11
CAPE = NCH * CAP_K
CWMAX = 16384
HH = 64
HT = 80
NEG_SLOPE = 0.2

PACK_R = 256
PACK_C = 3072
PACK_O = PACK_C // PBITS


def _pack_body(adj_ref, actr_ref, actc_ref, p_ref, out_ref):
    a = adj_ref[...]
    bits = jnp.where(
        (a > 0.0) & (actr_ref[...] > 0.0) & (actc_ref[...] > 0.0), 1.0, 0.0)
    out_ref[...] = lax.dot_general(
        bits.astype(jnp.bfloat16), p_ref[...],
        (((1,), (0,)), ((), ())), preferred_element_type=jnp.float32)


def _pack_call(adj, actr, actc, pmat):
    grid = (pl.cdiv(N, PACK_R), RW * PBITS // PACK_C)
    return pl.pallas_call(
        _pack_body,
        grid=grid,
        in_specs=[
            pl.BlockSpec((PACK_R, PACK_C), lambda r, c: (r, c)),
            pl.BlockSpec((PACK_R, 1), lambda r, c: (r, 0)),
            pl.BlockSpec((1, PACK_C), lambda r, c: (0, c)),
            pl.BlockSpec((PACK_C, PACK_O), lambda r, c: (0, 0)),
        ],
        out_specs=pl.BlockSpec((PACK_R, PACK_O), lambda r, c: (r, c)),
        out_shape=jax.ShapeDtypeStruct((N, RW), jnp.float32),
    )(adj, actr, actc, pmat)


def _mk_extract_body(wpt, cw, nchunks, roff):
  def _extract_body(bytes_hbm, src_hbm, dst_hbm, cnt_hbm,
                    chunk_v, chunk2_v, nzval_v, nzidx_v, srcbuf_v, dstbuf_v,
                    out16_v, sem, sem2):
    cid = lax.axis_index("c")
    sid = lax.axis_index("s")
    wid = sid * 2 + cid
    base = wid * wpt

    def _z(i, c):
        nzval_v[pl.ds(i * 16, 16)] = jnp.zeros((16,), jnp.float32)
        return c
    lax.fori_loop(0, NZCAP // 16, _z, 0)

    def _f(i, c):
        sent = N + ((i * 16) % (NPAD - N)) + lax.iota(jnp.int32, 16)
        srcbuf_v[pl.ds(i * 16, 16)] = sent
        dstbuf_v[pl.ds(i * 16, 16)] = sent
        return c
    lax.fori_loop(0, CAPE // 16, _f, 0)

    bufs = (chunk_v, chunk2_v)
    sems = (sem, sem2)
    hnd = [None, None]
    hnd[0] = pltpu.async_copy(bytes_hbm.at[pl.ds(base, cw)],
                              chunk_v.at[pl.ds(0, cw)], sem)
    nzcnt = jnp.int32(0)
    for ch in range(nchunks):
        p = ch % 2
        hnd[p].wait()
        if ch + 1 < nchunks:
            q = (ch + 1) % 2
            hnd[q] = pltpu.async_copy(
                bytes_hbm.at[pl.ds(base + (ch + 1) * cw, cw)],
                bufs[q].at[pl.ds(0, cw)], sems[q])
        buf = bufs[p]

        def _grp(j, o, buf=buf, ch=ch):
            vs = [buf[pl.ds((j * 4 + t) * 16, 16)] for t in range(4)]
            mx = jnp.maximum(jnp.maximum(vs[0], vs[1]),
                             jnp.maximum(vs[2], vs[3]))
            anynz = jnp.max(mx) > 0.0

            def _do(oo):
                for t in range(4):
                    v = vs[t]
                    m = v != 0.0
                    oo = jnp.minimum(oo, NZCAP - 16)
                    plsc.store_compressed(nzval_v.at[pl.ds(oo, 16)], v, mask=m)
                    idxv = (base + ch * cw + (j * 4 + t) * 16
                            + lax.iota(jnp.int32, 16)).astype(jnp.int32)
                    plsc.store_compressed(nzidx_v.at[pl.ds(oo, 16)], idxv,
                                          mask=m)
                    oo = oo + jnp.sum(m.astype(jnp.int32))
                return oo
            return lax.cond(anynz, _do, lambda oo: oo, o)
        nzcnt = lax.fori_loop(0, cw // 64, _grp, nzcnt)

    def _dec(q, eo):
        wv = nzval_v[pl.ds(q * 16, 16)]
        wi = nzidx_v[pl.ds(q * 16, 16)]
        w = wv.astype(jnp.int32)
        srcv = wi // RW + roff
        dstb = (wi % RW) * PBITS
        for b in range(PBITS):
            mb = ((w >> b) & 1) != 0
            eo = jnp.minimum(eo, CAPE - 16)
            plsc.store_compressed(srcbuf_v.at[pl.ds(eo, 16)], srcv, mask=mb)
            plsc.store_compressed(dstbuf_v.at[pl.ds(eo, 16)], dstb + b, mask=mb)
            eo = eo + jnp.sum(mb.astype(jnp.int32))
        return eo
    ecnt = lax.fori_loop(0, pl.cdiv(nzcnt, 16), _dec, jnp.int32(0))

    hnds = []
    for ch in range(NCH):
        hnds.append(pltpu.async_copy(srcbuf_v.at[pl.ds(ch * CAP_K, CAP_K)],
                                     src_hbm.at[wid, ch], sem))
        hnds.append(pltpu.async_copy(dstbuf_v.at[pl.ds(ch * CAP_K, CAP_K)],
                                     dst_hbm.at[wid, ch], sem2))
    out16_v[...] = jnp.broadcast_to(ecnt, (16,)).astype(jnp.int32)
    hnds.append(pltpu.async_copy(out16_v, cnt_hbm.at[wid], sem))
    for h in hnds:
        h.wait()
  return _extract_body


def _extract_call(bytes_flat):
    wpt = bytes_flat.shape[0] // NTILES
    cw = 16000
    mesh = plsc.VectorSubcoreMesh(core_axis_name="c", subcore_axis_name="s")
    f = pl.kernel(
        _mk_extract_body(wpt, cw, wpt // cw, 0),
        mesh=mesh,
        out_type=[
            jax.ShapeDtypeStruct((NTILES, NCH, CAP_K), jnp.int32),
            jax.ShapeDtypeStruct((NTILES, NCH, CAP_K), jnp.int32),
            jax.ShapeDtypeStruct((NTILES, 16), jnp.int32),
        ],
        scratch_types=[
            pltpu.VMEM((CWMAX,), jnp.float32),
            pltpu.VMEM((CWMAX,), jnp.float32),
            pltpu.VMEM((NZCAP,), jnp.float32),
            pltpu.VMEM((NZCAP,), jnp.int32),
            pltpu.VMEM((CAPE,), jnp.int32),
            pltpu.VMEM((CAPE,), jnp.int32),
            pltpu.VMEM((16,), jnp.int32),
            pltpu.SemaphoreType.DMA,
            pltpu.SemaphoreType.DMA,
        ],
        compiler_params=pltpu.CompilerParams(
            needs_layout_passes=False, use_tc_tiling_on_sc=False),
    )
    return f(bytes_flat)


def _msg_body(htab_hbm, as_hbm, ad_hbm, gmax_hbm, src0_hbm, dst0_hbm,
              cnt0_hbm,
              acc_out,
              vm_as, vm_ad, vm_g, src2d, dst2d, cnt16, rows_v, pbuf,
              zerob, acc_sh, sem):
    cid = lax.axis_index("c")
    sid = lax.axis_index("s")

    pltpu.sync_copy(as_hbm, vm_as)
    pltpu.sync_copy(ad_hbm, vm_ad)
    pltpu.sync_copy(gmax_hbm.at[pl.ds(0, 16)], vm_g)
    g = vm_g[...][0]

    def _zb(i, c):
        for k in range(HT // 16):
            zerob[i, pl.ds(k * 16, 16)] = jnp.zeros((16,), jnp.float32)
        return c
    lax.fori_loop(0, 64, _zb, 0)
    rows_per = NPAD // 16

    def _zs(r, c):
        pltpu.sync_copy(zerob, acc_sh.at[pl.ds(sid * rows_per + r * 64, 64), :])
        return c
    lax.fori_loop(0, rows_per // 64, _zs, 0)
    plsc.subcore_barrier()

    for src_hbm, dst_hbm, cnt_hbm in ((src0_hbm, dst0_hbm, cnt0_hbm),):
      for seg in range(2):
        wid = sid * 2 + seg
        pltpu.sync_copy(cnt_hbm.at[wid], cnt16)
        cnt = cnt16[...][0]
        pltpu.sync_copy(src_hbm.at[wid], src2d)
        pltpu.sync_copy(dst_hbm.at[wid], dst2d)

        def _chunk(ch, c):
            pltpu.async_copy(htab_hbm.at[cid].at[src2d.at[ch]], rows_v,
                             sem).wait()
            for i in range(CAP_K // 16):
                sv = src2d[ch, pl.ds(i * 16, 16)]
                dv = dst2d[ch, pl.ds(i * 16, 16)]
                asg = plsc.load_gather(vm_as, [sv])
                adg = plsc.load_gather(vm_ad, [dv])
                mg = g + adg
                mg = jnp.where(mg >= 0.0, mg, NEG_SLOPE * mg)
                z = asg + adg
                z = jnp.where(z >= 0.0, z, NEG_SLOPE * z)
                pbuf[pl.ds(i * 16, 16)] = jnp.exp(z - mg)

            def _scale(q, cc):
                p16 = pbuf[pl.ds(q * 16, 16)]
                for i in range(16):
                    r = q * 16 + i
                    pr = p16[i]
                    for k in range(HT // 16):
                        rows_v[r, pl.ds(k * 16, 16)] = (
                            rows_v[r, pl.ds(k * 16, 16)] * pr)
                return cc
            lax.fori_loop(0, CAP_K // 16, _scale, 0)
            pltpu.sync_copy(rows_v, acc_sh.at[dst2d.at[ch]], add=True)
            return c
        lax.fori_loop(0, pl.cdiv(cnt, CAP_K), _chunk, 0)

    plsc.subcore_barrier()
    pltpu.sync_copy(acc_sh.at[pl.ds(sid * rows_per, rows_per), :],
                    acc_out.at[cid, pl.ds(sid * rows_per, rows_per), :])


def _msg_call(htab, asf, adf, gmaxf, src0, dst0, cnt0):
    mesh = plsc.VectorSubcoreMesh(core_axis_name="c", subcore_axis_name="s")
    f = pl.kernel(
        _msg_body,
        mesh=mesh,
        out_type=[jax.ShapeDtypeStruct((2, NPAD, HT), jnp.float32)],
        scratch_types=[
            pltpu.VMEM((NPAD,), jnp.float32),
            pltpu.VMEM((NPAD,), jnp.float32),
            pltpu.VMEM((16,), jnp.float32),
            pltpu.VMEM((NCH, CAP_K), jnp.int32),
            pltpu.VMEM((NCH, CAP_K), jnp.int32),
            pltpu.VMEM((16,), jnp.int32),
            pltpu.VMEM((CAP_K, HT), jnp.float32),
            pltpu.VMEM((CAP_K,), jnp.float32),
            pltpu.VMEM((64, HT), jnp.float32),
            pltpu.VMEM_SHARED((NPAD, HT), jnp.float32),
            pltpu.SemaphoreType.DMA,
        ],
        compiler_params=pltpu.CompilerParams(
            needs_layout_passes=False, use_tc_tiling_on_sc=False),
    )
    (acc,) = f(htab, asf, adf, gmaxf, src0, dst0, cnt0)
    return acc


def _head(x, gw_ref, atts_ref, attd_ref, b, htab_ref, as_ref, ad_ref,
          gmax_ref, rowmask):
    h = lax.dot_general(x, gw_ref[...], (((1,), (1,)), ((), ())),
                        preferred_element_type=jnp.float32)
    a_s = jnp.sum(h * atts_ref[...], axis=1, keepdims=True)
    a_d = jnp.sum(h * attd_ref[...], axis=1, keepdims=True)
    onescol = rowmask.astype(jnp.float32)
    zpad = jnp.zeros((TCR, HT - HH - 1), jnp.float32)
    htab_ref[0] = jnp.concatenate([h[:, 0:HH], onescol, zpad], axis=1)
    htab_ref[1] = jnp.concatenate([h[:, HH:H], onescol, zpad], axis=1)
    as_ref[...] = a_s
    ad_ref[...] = a_d

    @pl.when(b == 0)
    def _():
        gmax_ref[...] = jnp.full((1, H), -jnp.inf, jnp.float32)
    gmax_ref[...] = jnp.maximum(gmax_ref[...], jnp.max(a_s))


def _a0_body(prog_ref, hard_ref, w1_ref, b1_ref, w2_ref, b2_ref,
             gw_ref, atts_ref, attd_ref,
             x0_ref, htab_ref, as_ref, ad_ref, gmax_ref):
    b = pl.program_id(0)
    in2 = jnp.concatenate([prog_ref[...], hard_ref[...]], axis=1)
    t1 = lax.dot_general(in2, w1_ref[...], (((1,), (1,)), ((), ())),
                         preferred_element_type=jnp.float32) + b1_ref[...]
    x0 = lax.dot_general(t1, w2_ref[...], (((1,), (1,)), ((), ())),
                         preferred_element_type=jnp.float32) + b2_ref[...]
    rowmask = (lax.broadcasted_iota(jnp.int32, (TCR, 1), 0) + b * TCR) < N
    x0 = jnp.where(rowmask, x0, 0.0)
    x0_ref[...] = x0
    _head(x0, gw_ref, atts_ref, attd_ref, b, htab_ref, as_ref, ad_ref,
          gmax_ref, rowmask)


def _combine(acc_ref, htabp_ref, asp_ref, adp_ref, gmaxp_ref, bias_ref,
             apply_relu, rowmask):
    num = jnp.concatenate([acc_ref[0][:, 0:HH], acc_ref[1][:, 0:HH]], axis=1)
    den = acc_ref[0][:, HH:HH + 1]
    hprev = jnp.concatenate([htabp_ref[0][:, 0:HH], htabp_ref[1][:, 0:HH]],
                            axis=1)
    asb = asp_ref[...]
    adb = adp_ref[...]
    g = gmaxp_ref[0, 0]
    m = g + adb
    mm = jnp.where(m >= 0.0, m, NEG_SLOPE * m)
    zs = asb + adb
    zs = jnp.where(zs >= 0.0, zs, NEG_SLOPE * zs)
    p_self = jnp.exp(zs - mm)
    out = (num + p_self * hprev) / (den + p_self + 1e-16) + bias_ref[...]
    if apply_relu:
        out = jnp.maximum(out, 0.0)
    return jnp.where(rowmask, out, 0.0)


def _mk_anext_body(apply_relu):
    def _body(acc_ref, htabp_ref, asp_ref, adp_ref, gmaxp_ref, bias_ref,
              gw_ref, atts_ref, attd_ref,
              htab_ref, as_ref, ad_ref, gmax_ref):
        b = pl.program_id(0)
        rowmask = (lax.broadcasted_iota(jnp.int32, (TCR, 1), 0) + b * TCR) < N
        x = _combine(acc_ref, htabp_ref, asp_ref, adp_ref, gmaxp_ref,
                     bias_ref, apply_relu, rowmask)
        _head(x, gw_ref, atts_ref, attd_ref, b, htab_ref, as_ref, ad_ref,
              gmax_ref, rowmask)
    return _body


def _fin_body(acc_ref, htabp_ref, asp_ref, adp_ref, gmaxp_ref, bias_ref,
              x0_ref, act_ref, vw_ref, vb_ref,
              pooled_ref, na_ref, v_ref):
    b = pl.program_id(0)
    rowmask = (lax.broadcasted_iota(jnp.int32, (TCR, 1), 0) + b * TCR) < N
    gat = _combine(acc_ref, htabp_ref, asp_ref, adp_ref, gmaxp_ref,
                   bias_ref, False, rowmask)
    xf = x0_ref[...] + gat
    mu = jnp.mean(xf, axis=1, keepdims=True)
    var = jnp.mean((xf - mu) * (xf - mu), axis=1, keepdims=True)
    xn = (xf - mu) / jnp.sqrt(var + 1e-5)
    actb = act_ref[...]

    @pl.when(b == 0)
    def _():
        pooled_ref[...] = jnp.zeros((1, H), jnp.float32)
        na_ref[...] = jnp.zeros((1, H), jnp.float32)
    pooled_ref[...] = pooled_ref[...] + jnp.sum(xn * actb, axis=0,
                                                keepdims=True)
    na_ref[...] = na_ref[...] + jnp.sum(actb)

    @pl.when(b == NBLK - 1)
    def _():
        pooled = pooled_ref[...] / na_ref[0, 0]
        v = jnp.sum(pooled * vw_ref[...]) + vb_ref[0, 0]
        v_ref[...] = jnp.maximum(v, 0.0).reshape(1, 1)


_SPEC_COL = pl.BlockSpec((TCR, 1), lambda b: (b, 0))
_SPEC_ROW = pl.BlockSpec((TCR, H), lambda b: (b, 0))
_SPEC_HTAB = pl.BlockSpec((2, TCR, HT), lambda b: (0, b, 0))
_SPEC_FULL = pl.BlockSpec((H, H), lambda b: (0, 0))
_SPEC_VEC = pl.BlockSpec((1, H), lambda b: (0, 0))
_SPEC_ACC = pl.BlockSpec((2, TCR, HT), lambda b: (0, b, 0))

_HEAD_OUT_SPECS = [_SPEC_HTAB, _SPEC_COL, _SPEC_COL, _SPEC_VEC]


def _head_out_shapes():
    return [
        jax.ShapeDtypeStruct((2, NPAD, HT), jnp.float32),
        jax.ShapeDtypeStruct((NPAD, 1), jnp.float32),
        jax.ShapeDtypeStruct((NPAD, 1), jnp.float32),
        jax.ShapeDtypeStruct((1, H), jnp.float32),
    ]


def _a0_call(prog, hard, w1, b1, w2, b2, gw, atts, attd):
    return pl.pallas_call(
        _a0_body,
        grid=(NBLK,),
        in_specs=[_SPEC_COL, _SPEC_COL,
                  pl.BlockSpec((H, 2), lambda b: (0, 0)), _SPEC_VEC,
                  _SPEC_FULL, _SPEC_VEC, _SPEC_FULL, _SPEC_VEC, _SPEC_VEC],
        out_specs=[_SPEC_ROW] + _HEAD_OUT_SPECS,
        out_shape=[jax.ShapeDtypeStruct((NPAD, H), jnp.float32)]
        + _head_out_shapes(),
    )(prog, hard, w1, b1, w2, b2, gw, atts, attd)


def _anext_call(apply_relu, acc, htabp, asp, adp, gmaxp, bias, gw, atts, attd):
    return pl.pallas_call(
        _mk_anext_body(apply_relu),
        grid=(NBLK,),
        in_specs=[_SPEC_ACC, _SPEC_HTAB, _SPEC_COL, _SPEC_COL, _SPEC_VEC,
                  _SPEC_VEC, _SPEC_FULL, _SPEC_VEC, _SPEC_VEC],
        out_specs=_HEAD_OUT_SPECS,
        out_shape=_head_out_shapes(),
    )(acc, htabp, asp, adp, gmaxp, bias, gw, atts, attd)


def _fin_call(acc, htabp, asp, adp, gmaxp, bias, x0, act, vw, vb):
    return pl.pallas_call(
        _fin_body,
        grid=(NBLK,),
        in_specs=[_SPEC_ACC, _SPEC_HTAB, _SPEC_COL, _SPEC_COL, _SPEC_VEC,
                  _SPEC_VEC, _SPEC_ROW, _SPEC_COL, _SPEC_VEC,
                  pl.BlockSpec((1, 1), lambda b: (0, 0))],
        out_specs=[_SPEC_VEC, _SPEC_VEC, pl.BlockSpec((1, 1), lambda b: (0, 0))],
        out_shape=[
            jax.ShapeDtypeStruct((1, H), jnp.float32),
            jax.ShapeDtypeStruct((1, H), jnp.float32),
            jax.ShapeDtypeStruct((1, 1), jnp.float32),
        ],
    )(acc, htabp, asp, adp, gmaxp, bias, x0, act, vw, vb)


def _padcol(a):
    return jnp.pad(a.astype(jnp.float32), (0, NPAD - N)).reshape(NPAD, 1)


def kernel(adjacency_matrix, timestep, arrivals, departures, is_hard_to_match,
           active_agents, emb_W1, emb_b1, emb_W2, emb_b2, gat_W, gat_att_src,
           gat_att_dst, gat_bias, value_W, value_b):
    t = jnp.asarray(timestep, jnp.float32)
    prog = (t - arrivals) / (departures - arrivals)
    progc = _padcol(prog)
    hardc = _padcol(is_hard_to_match)
    actc = _padcol(active_agents)
    actr = active_agents.reshape(N, 1)
    actcol = jnp.pad(active_agents, (0, RW * PBITS - N)).reshape(1, RW * PBITS)

    ki = lax.broadcasted_iota(jnp.int32, (PACK_C, PACK_O), 0)
    gi = lax.broadcasted_iota(jnp.int32, (PACK_C, PACK_O), 1)
    pmat = jnp.where(ki // PBITS == gi,
                     (1 << (ki % PBITS)).astype(jnp.float32),
                     0.0).astype(jnp.bfloat16)

    bytes0 = _pack_call(adjacency_matrix, actr, actcol, pmat)
    src0, dst0, cnt0 = _extract_call(bytes0.reshape(-1))

    x0, htab0, as0, ad0, g0 = _a0_call(
        progc, hardc, emb_W1, emb_b1.reshape(1, H), emb_W2,
        emb_b2.reshape(1, H), gat_W[0], gat_att_src[0].reshape(1, H),
        gat_att_dst[0].reshape(1, H))

    htab, asp, adp, gp = htab0, as0, ad0, g0
    for l in range(3):
        acc = _msg_call(htab, asp.reshape(NPAD), adp.reshape(NPAD),
                        gp.reshape(H), src0, dst0, cnt0)
        bias = gat_bias[l].reshape(1, H)
        if l < 2:
            htab, asp, adp, gp = _anext_call(
                True, acc, htab, asp, adp, gp, bias, gat_W[l + 1],
                gat_att_src[l + 1].reshape(1, H),
                gat_att_dst[l + 1].reshape(1, H))
        else:
            _, _, v = _fin_call(acc, htab, asp, adp, gp, bias, x0, actc,
                                value_W.reshape(1, H), value_b.reshape(1, 1))
    return v[0, 0]


# speedup vs baseline: 1.0432x; 1.0004x over previous
import jax
import jax.numpy as jnp
from jax import lax
from jax.experimental import pallas as pl
from jax.experimental.pallas import tpu as pltpu
from jax.experimental.pallas import tpu_sc as plsc

N = 10000
H = 128
NPAD = 10240
TCR = 512
NBLK = NPAD // TCR
SENT = NPAD - 1
PBITS = 24
RW = 512
NTILES = 32
NZCAP = 8192
CAP_K = 384
NCH = 11
CAPE = NCH * CAP_K
CWMAX = 16384
HH = 64
HT = 80
NEG_SLOPE = 0.2

PACK_R = 256
PACK_C = 3072
PACK_O = PACK_C // PBITS


def _pack_body(adj_ref, actr_ref, actc_ref, p_ref, out_ref):
    a = adj_ref[...]
    bits = jnp.where(
        (a > 0.0) & (actr_ref[...] > 0.0) & (actc_ref[...] > 0.0), 1.0, 0.0)
    out_ref[...] = lax.dot_general(
        bits.astype(jnp.bfloat16), p_ref[...],
        (((1,), (0,)), ((), ())), preferred_element_type=jnp.float32)


def _pack_call(adj, actr, actc, pmat):
    grid = (pl.cdiv(N, PACK_R), RW * PBITS // PACK_C)
    return pl.pallas_call(
        _pack_body,
        grid=grid,
        in_specs=[
            pl.BlockSpec((PACK_R, PACK_C), lambda r, c: (r, c)),
            pl.BlockSpec((PACK_R, 1), lambda r, c: (r, 0)),
            pl.BlockSpec((1, PACK_C), lambda r, c: (0, c)),
            pl.BlockSpec((PACK_C, PACK_O), lambda r, c: (0, 0)),
        ],
        out_specs=pl.BlockSpec((PACK_R, PACK_O), lambda r, c: (r, c)),
        out_shape=jax.ShapeDtypeStruct((N, RW), jnp.float32),
    )(adj, actr, actc, pmat)


def _mk_extract_body(wpt, cw, nchunks, roff):
  def _extract_body(bytes_hbm, src_hbm, dst_hbm, cnt_hbm,
                    chunk_v, chunk2_v, nzval_v, nzidx_v, srcbuf_v, dstbuf_v,
                    out16_v, sem, sem2):
    cid = lax.axis_index("c")
    sid = lax.axis_index("s")
    wid = sid * 2 + cid
    base = wid * wpt

    def _z(i, c):
        nzval_v[pl.ds(i * 16, 16)] = jnp.zeros((16,), jnp.float32)
        return c
    lax.fori_loop(0, NZCAP // 16, _z, 0)

    def _f(i, c):
        sent = N + ((i * 16) % (NPAD - N)) + lax.iota(jnp.int32, 16)
        srcbuf_v[pl.ds(i * 16, 16)] = sent
        dstbuf_v[pl.ds(i * 16, 16)] = sent
        return c
    lax.fori_loop(0, CAPE // 16, _f, 0)

    bufs = (chunk_v, chunk2_v)
    sems = (sem, sem2)
    hnd = [None, None]
    hnd[0] = pltpu.async_copy(bytes_hbm.at[pl.ds(base, cw)],
                              chunk_v.at[pl.ds(0, cw)], sem)
    nzcnt = jnp.int32(0)
    for ch in range(nchunks):
        p = ch % 2
        hnd[p].wait()
        if ch + 1 < nchunks:
            q = (ch + 1) % 2
            hnd[q] = pltpu.async_copy(
                bytes_hbm.at[pl.ds(base + (ch + 1) * cw, cw)],
                bufs[q].at[pl.ds(0, cw)], sems[q])
        buf = bufs[p]

        def _grp(j, o, buf=buf, ch=ch):
            vs = [buf[pl.ds((j * 4 + t) * 16, 16)] for t in range(4)]
            mx = jnp.maximum(jnp.maximum(vs[0], vs[1]),
                             jnp.maximum(vs[2], vs[3]))
            anynz = jnp.max(mx) > 0.0

            def _do(oo):
                for t in range(4):
                    v = vs[t]
                    m = v != 0.0
                    oo = jnp.minimum(oo, NZCAP - 16)
                    plsc.store_compressed(nzval_v.at[pl.ds(oo, 16)], v, mask=m)
                    idxv = (base + ch * cw + (j * 4 + t) * 16
                            + lax.iota(jnp.int32, 16)).astype(jnp.int32)
                    plsc.store_compressed(nzidx_v.at[pl.ds(oo, 16)], idxv,
                                          mask=m)
                    oo = oo + jnp.sum(m.astype(jnp.int32))
                return oo
            return lax.cond(anynz, _do, lambda oo: oo, o)
        nzcnt = lax.fori_loop(0, cw // 64, _grp, nzcnt)

    def _dec(q, eo):
        wv = nzval_v[pl.ds(q * 16, 16)]
        wi = nzidx_v[pl.ds(q * 16, 16)]
        w = wv.astype(jnp.int32)
        srcv = wi // RW + roff
        dstb = (wi % RW) * PBITS
        for b in range(PBITS):
            mb = ((w >> b) & 1) != 0
            eo = jnp.minimum(eo, CAPE - 16)
            plsc.store_compressed(srcbuf_v.at[pl.ds(eo, 16)], srcv, mask=mb)
            plsc.store_compressed(dstbuf_v.at[pl.ds(eo, 16)], dstb + b, mask=mb)
            eo = eo + jnp.sum(mb.astype(jnp.int32))
        return eo
    ecnt = lax.fori_loop(0, pl.cdiv(nzcnt, 16), _dec, jnp.int32(0))

    hnds = []
    for ch in range(NCH):
        hnds.append(pltpu.async_copy(srcbuf_v.at[pl.ds(ch * CAP_K, CAP_K)],
                                     src_hbm.at[wid, ch], sem))
        hnds.append(pltpu.async_copy(dstbuf_v.at[pl.ds(ch * CAP_K, CAP_K)],
                                     dst_hbm.at[wid, ch], sem2))
    out16_v[...] = jnp.broadcast_to(ecnt, (16,)).astype(jnp.int32)
    hnds.append(pltpu.async_copy(out16_v, cnt_hbm.at[wid], sem))
    for h in hnds:
        h.wait()
  return _extract_body


def _extract_call(bytes_flat):
    wpt = bytes_flat.shape[0] // NTILES
    cw = 16000
    mesh = plsc.VectorSubcoreMesh(core_axis_name="c", subcore_axis_name="s")
    f = pl.kernel(
        _mk_extract_body(wpt, cw, wpt // cw, 0),
        mesh=mesh,
        out_type=[
            jax.ShapeDtypeStruct((NTILES, NCH, CAP_K), jnp.int32),
            jax.ShapeDtypeStruct((NTILES, NCH, CAP_K), jnp.int32),
            jax.ShapeDtypeStruct((NTILES, 16), jnp.int32),
        ],
        scratch_types=[
            pltpu.VMEM((CWMAX,), jnp.float32),
            pltpu.VMEM((CWMAX,), jnp.float32),
            pltpu.VMEM((NZCAP,), jnp.float32),
            pltpu.VMEM((NZCAP,), jnp.int32),
            pltpu.VMEM((CAPE,), jnp.int32),
            pltpu.VMEM((CAPE,), jnp.int32),
            pltpu.VMEM((16,), jnp.int32),
            pltpu.SemaphoreType.DMA,
            pltpu.SemaphoreType.DMA,
        ],
        compiler_params=pltpu.CompilerParams(
            needs_layout_passes=False, use_tc_tiling_on_sc=False),
    )
    return f(bytes_flat)


def _msg_body(htab_hbm, as_hbm, ad_hbm, gmax_hbm, src0_hbm, dst0_hbm,
              cnt0_hbm,
              acc_out,
              vm_as, vm_ad, vm_g, src2d, dst2d, cnt16, rows_v, pbuf,
              zerob, acc_sh, sem):
    cid = lax.axis_index("c")
    sid = lax.axis_index("s")

    pltpu.sync_copy(as_hbm, vm_as)
    pltpu.sync_copy(ad_hbm, vm_ad)
    pltpu.sync_copy(gmax_hbm.at[pl.ds(0, 16)], vm_g)
    g = vm_g[...][0]

    def _zb(i, c):
        for k in range(HT // 16):
            zerob[i, pl.ds(k * 16, 16)] = jnp.zeros((16,), jnp.float32)
        return c
    lax.fori_loop(0, 64, _zb, 0)
    rows_per = NPAD // 16

    def _zs(r, c):
        pltpu.sync_copy(zerob, acc_sh.at[pl.ds(sid * rows_per + r * 64, 64), :])
        return c
    lax.fori_loop(0, rows_per // 64, _zs, 0)
    plsc.subcore_barrier()

    for src_hbm, dst_hbm, cnt_hbm in ((src0_hbm, dst0_hbm, cnt0_hbm),):
      for seg in range(2):
        wid = sid * 2 + seg
        pltpu.sync_copy(cnt_hbm.at[wid], cnt16)
        cnt = cnt16[...][0]
        pltpu.sync_copy(src_hbm.at[wid], src2d)
        pltpu.sync_copy(dst_hbm.at[wid], dst2d)

        def _chunk(ch, c):
            pltpu.async_copy(htab_hbm.at[cid].at[src2d.at[ch]], rows_v,
                             sem).wait()
            for i in range(CAP_K // 16):
                sv = src2d[ch, pl.ds(i * 16, 16)]
                dv = dst2d[ch, pl.ds(i * 16, 16)]
                asg = plsc.load_gather(vm_as, [sv])
                adg = plsc.load_gather(vm_ad, [dv])
                mg = g + adg
                mg = jnp.where(mg >= 0.0, mg, NEG_SLOPE * mg)
                z = asg + adg
                z = jnp.where(z >= 0.0, z, NEG_SLOPE * z)
                pbuf[pl.ds(i * 16, 16)] = jnp.exp(z - mg)

            def _scale(q, cc):
                p16 = pbuf[pl.ds(q * 16, 16)]
                for i in range(16):
                    r = q * 16 + i
                    pr = p16[i]
                    for k in range(HT // 16):
                        rows_v[r, pl.ds(k * 16, 16)] = (
                            rows_v[r, pl.ds(k * 16, 16)] * pr)
                return cc
            lax.fori_loop(0, CAP_K // 16, _scale, 0)
            pltpu.sync_copy(rows_v, acc_sh.at[dst2d.at[ch]], add=True)
            return c
        lax.fori_loop(0, pl.cdiv(cnt, CAP_K), _chunk, 0)

    plsc.subcore_barrier()
    pltpu.sync_copy(acc_sh.at[pl.ds(sid * rows_per, rows_per), :],
                    acc_out.at[cid, pl.ds(sid * rows_per, rows_per), :])


def _msg_call(htab, asf, adf, gmaxf, src0, dst0, cnt0):
    mesh = plsc.VectorSubcoreMesh(core_axis_name="c", subcore_axis_name="s")
    f = pl.kernel(
        _msg_body,
        mesh=mesh,
        out_type=[jax.ShapeDtypeStruct((2, NPAD, HT), jnp.float32)],
        scratch_types=[
            pltpu.VMEM((NPAD,), jnp.float32),
            pltpu.VMEM((NPAD,), jnp.float32),
            pltpu.VMEM((16,), jnp.float32),
            pltpu.VMEM((NCH, CAP_K), jnp.int32),
            pltpu.VMEM((NCH, CAP_K), jnp.int32),
            pltpu.VMEM((16,), jnp.int32),
            pltpu.VMEM((CAP_K, HT), jnp.float32),
            pltpu.VMEM((CAP_K,), jnp.float32),
            pltpu.VMEM((64, HT), jnp.float32),
            pltpu.VMEM_SHARED((NPAD, HT), jnp.float32),
            pltpu.SemaphoreType.DMA,
        ],
        compiler_params=pltpu.CompilerParams(
            needs_layout_passes=False, use_tc_tiling_on_sc=False),
    )
    (acc,) = f(htab, asf, adf, gmaxf, src0, dst0, cnt0)
    return acc


def _head(x, gw_ref, atts_ref, attd_ref, b, htab_ref, as_ref, ad_ref,
          gmax_ref, rowmask):
    h = lax.dot_general(x, gw_ref[...], (((1,), (1,)), ((), ())),
                        preferred_element_type=jnp.float32)
    a_s = jnp.sum(h * atts_ref[...], axis=1, keepdims=True)
    a_d = jnp.sum(h * attd_ref[...], axis=1, keepdims=True)
    onescol = rowmask.astype(jnp.float32)
    zpad = jnp.zeros((TCR, HT - HH - 1), jnp.float32)
    htab_ref[0] = jnp.concatenate([h[:, 0:HH], onescol, zpad], axis=1)
    htab_ref[1] = jnp.concatenate([h[:, HH:H], onescol, zpad], axis=1)
    as_ref[...] = a_s
    ad_ref[...] = a_d

    @pl.when(b == 0)
    def _():
        gmax_ref[...] = jnp.full((1, H), -jnp.inf, jnp.float32)
    gmax_ref[...] = jnp.maximum(gmax_ref[...], jnp.max(a_s))


def _a0_body(prog_ref, hard_ref, w1_ref, b1_ref, w2_ref, b2_ref,
             gw_ref, atts_ref, attd_ref,
             x0_ref, htab_ref, as_ref, ad_ref, gmax_ref):
    b = pl.program_id(0)
    in2 = jnp.concatenate([prog_ref[...], hard_ref[...]], axis=1)
    t1 = lax.dot_general(in2, w1_ref[...], (((1,), (1,)), ((), ())),
                         preferred_element_type=jnp.float32) + b1_ref[...]
    x0 = lax.dot_general(t1, w2_ref[...], (((1,), (1,)), ((), ())),
                         preferred_element_type=jnp.float32) + b2_ref[...]
    rowmask = (lax.broadcasted_iota(jnp.int32, (TCR, 1), 0) + b * TCR) < N
    x0 = jnp.where(rowmask, x0, 0.0)
    x0_ref[...] = x0
    _head(x0, gw_ref, atts_ref, attd_ref, b, htab_ref, as_ref, ad_ref,
          gmax_ref, rowmask)


def _combine(acc_ref, htabp_ref, asp_ref, adp_ref, gmaxp_ref, bias_ref,
             apply_relu, rowmask):
    num = jnp.concatenate([acc_ref[0][:, 0:HH], acc_ref[1][:, 0:HH]], axis=1)
    den = acc_ref[0][:, HH:HH + 1]
    hprev = jnp.concatenate([htabp_ref[0][:, 0:HH], htabp_ref[1][:, 0:HH]],
                            axis=1)
    asb = asp_ref[...]
    adb = adp_ref[...]
    g = gmaxp_ref[0, 0]
    m = g + adb
    mm = jnp.where(m >= 0.0, m, NEG_SLOPE * m)
    zs = asb + adb
    zs = jnp.where(zs >= 0.0, zs, NEG_SLOPE * zs)
    p_self = jnp.exp(zs - mm)
    out = (num + p_self * hprev) / (den + p_self + 1e-16) + bias_ref[...]
    if apply_relu:
        out = jnp.maximum(out, 0.0)
    return jnp.where(rowmask, out, 0.0)


def _mk_anext_body(apply_relu):
    def _body(acc_ref, htabp_ref, asp_ref, adp_ref, gmaxp_ref, bias_ref,
              gw_ref, atts_ref, attd_ref,
              htab_ref, as_ref, ad_ref, gmax_ref):
        b = pl.program_id(0)
        rowmask = (lax.broadcasted_iota(jnp.int32, (TCR, 1), 0) + b * TCR) < N
        x = _combine(acc_ref, htabp_ref, asp_ref, adp_ref, gmaxp_ref,
                     bias_ref, apply_relu, rowmask)
        _head(x, gw_ref, atts_ref, attd_ref, b, htab_ref, as_ref, ad_ref,
              gmax_ref, rowmask)
    return _body


def _fin_body(acc_ref, htabp_ref, asp_ref, adp_ref, gmaxp_ref, bias_ref,
              x0_ref, act_ref, vw_ref, vb_ref,
              pooled_ref, na_ref, v_ref):
    b = pl.program_id(0)
    rowmask = (lax.broadcasted_iota(jnp.int32, (TCR, 1), 0) + b * TCR) < N
    gat = _combine(acc_ref, htabp_ref, asp_ref, adp_ref, gmaxp_ref,
                   bias_ref, False, rowmask)
    xf = x0_ref[...] + gat
    mu = jnp.mean(xf, axis=1, keepdims=True)
    var = jnp.mean((xf - mu) * (xf - mu), axis=1, keepdims=True)
    xn = (xf - mu) / jnp.sqrt(var + 1e-5)
    actb = act_ref[...]

    @pl.when(b == 0)
    def _():
        pooled_ref[...] = jnp.zeros((1, H), jnp.float32)
        na_ref[...] = jnp.zeros((1, H), jnp.float32)
    pooled_ref[...] = pooled_ref[...] + jnp.sum(xn * actb, axis=0,
                                                keepdims=True)
    na_ref[...] = na_ref[...] + jnp.sum(actb)

    @pl.when(b == NBLK - 1)
    def _():
        pooled = pooled_ref[...] / na_ref[0, 0]
        v = jnp.sum(pooled * vw_ref[...]) + vb_ref[0, 0]
        v_ref[...] = jnp.maximum(v, 0.0).reshape(1, 1)


_SPEC_COL = pl.BlockSpec((TCR, 1), lambda b: (b, 0))
_SPEC_ROW = pl.BlockSpec((TCR, H), lambda b: (b, 0))
_SPEC_HTAB = pl.BlockSpec((2, TCR, HT), lambda b: (0, b, 0))
_SPEC_FULL = pl.BlockSpec((H, H), lambda b: (0, 0))
_SPEC_VEC = pl.BlockSpec((1, H), lambda b: (0, 0))
_SPEC_ACC = pl.BlockSpec((2, TCR, HT), lambda b: (0, b, 0))

_HEAD_OUT_SPECS = [_SPEC_HTAB, _SPEC_COL, _SPEC_COL, _SPEC_VEC]


def _head_out_shapes():
    return [
        jax.ShapeDtypeStruct((2, NPAD, HT), jnp.float32),
        jax.ShapeDtypeStruct((NPAD, 1), jnp.float32),
        jax.ShapeDtypeStruct((NPAD, 1), jnp.float32),
        jax.ShapeDtypeStruct((1, H), jnp.float32),
    ]


def _a0_call(prog, hard, w1, b1, w2, b2, gw, atts, attd):
    return pl.pallas_call(
        _a0_body,
        grid=(NBLK,),
        in_specs=[_SPEC_COL, _SPEC_COL,
                  pl.BlockSpec((H, 2), lambda b: (0, 0)), _SPEC_VEC,
                  _SPEC_FULL, _SPEC_VEC, _SPEC_FULL, _SPEC_VEC, _SPEC_VEC],
        out_specs=[_SPEC_ROW] + _HEAD_OUT_SPECS,
        out_shape=[jax.ShapeDtypeStruct((NPAD, H), jnp.float32)]
        + _head_out_shapes(),
    )(prog, hard, w1, b1, w2, b2, gw, atts, attd)


def _anext_call(apply_relu, acc, htabp, asp, adp, gmaxp, bias, gw, atts, attd):
    return pl.pallas_call(
        _mk_anext_body(apply_relu),
        grid=(NBLK,),
        in_specs=[_SPEC_ACC, _SPEC_HTAB, _SPEC_COL, _SPEC_COL, _SPEC_VEC,
                  _SPEC_VEC, _SPEC_FULL, _SPEC_VEC, _SPEC_VEC],
        out_specs=_HEAD_OUT_SPECS,
        out_shape=_head_out_shapes(),
    )(acc, htabp, asp, adp, gmaxp, bias, gw, atts, attd)


def _fin_call(acc, htabp, asp, adp, gmaxp, bias, x0, act, vw, vb):
    return pl.pallas_call(
        _fin_body,
        grid=(NBLK,),
        in_specs=[_SPEC_ACC, _SPEC_HTAB, _SPEC_COL, _SPEC_COL, _SPEC_VEC,
                  _SPEC_VEC, _SPEC_ROW, _SPEC_COL, _SPEC_VEC,
                  pl.BlockSpec((1, 1), lambda b: (0, 0))],
        out_specs=[_SPEC_VEC, _SPEC_VEC, pl.BlockSpec((1, 1), lambda b: (0, 0))],
        out_shape=[
            jax.ShapeDtypeStruct((1, H), jnp.float32),
            jax.ShapeDtypeStruct((1, H), jnp.float32),
            jax.ShapeDtypeStruct((1, 1), jnp.float32),
        ],
    )(acc, htabp, asp, adp, gmaxp, bias, x0, act, vw, vb)


def _padcol(a):
    return jnp.pad(a.astype(jnp.float32), (0, NPAD - N)).reshape(NPAD, 1)


def kernel(adjacency_matrix, timestep, arrivals, departures, is_hard_to_match,
           active_agents, emb_W1, emb_b1, emb_W2, emb_b2, gat_W, gat_att_src,
           gat_att_dst, gat_bias, value_W, value_b):
    t = jnp.asarray(timestep, jnp.float32)
    prog = (t - arrivals) / (departures - arrivals)
    progc = _padcol(prog)
    hardc = _padcol(is_hard_to_match)
    actc = _padcol(active_agents)
    actr = active_agents.reshape(N, 1)
    actcol = jnp.pad(active_agents, (0, RW * PBITS - N)).reshape(1, RW * PBITS)

    ki = lax.broadcasted_iota(jnp.int32, (PACK_C, PACK_O), 0)
    gi = lax.broadcasted_iota(jnp.int32, (PACK_C, PACK_O), 1)
    pmat = jnp.where(ki // PBITS == gi,
                     (1 << (ki % PBITS)).astype(jnp.float32),
                     0.0).astype(jnp.bfloat16)

    bytes0 = _pack_call(adjacency_matrix, actr, actcol, pmat)
    src0, dst0, cnt0 = _extract_call(bytes0.reshape(-1))

    x0, htab0, as0, ad0, g0 = _a0_call(
        progc, hardc, emb_W1, emb_b1.reshape(1, H), emb_W2,
        emb_b2.reshape(1, H), gat_W[0], gat_att_src[0].reshape(1, H),
        gat_att_dst[0].reshape(1, H))

    htab, asp, adp, gp = htab0, as0, ad0, g0
    for l in range(3):
        acc = _msg_call(htab, asp.reshape(NPAD), adp.reshape(NPAD),
                        gp.reshape(H), src0, dst0, cnt0)
        bias = gat_bias[l].reshape(1, H)
        if l < 2:
            htab, asp, adp, gp = _anext_call(
                True, acc, htab, asp, adp, gp, bias, gat_W[l + 1],
                gat_att_src[l + 1].reshape(1, H),
                gat_att_dst[l + 1].reshape(1, H))
        else:
            _, _, v = _fin_call(acc, htab, asp, adp, gp, bias, x0, actc,
                                value_W.reshape(1, H), value_b.reshape(1, 1))
    return v[0, 0]
